# baseline (speedup 1.0000x reference)
import functools

import jax
import jax.numpy as jnp
from jax import lax
from jax.experimental import pallas as pl
from jax.experimental.pallas import tpu as pltpu
from jax.experimental.pallas import tpu_sc as plsc

_NC = 2
_NS = 16
_NW = _NC * _NS

_N = 4096
_S = 2048
_ND = 4
_NCONT = 32
_DIM = 512
_L = 16
_NK = _DIM // _L
_NROW = _ND + _NCONT
_HEAD = 8
_TAIL = _NROW - _HEAD
_PW = _N // _NW
_PP = 2
_STEPS = _PW // _PP
_G = 4


def _sc_body(idx_hbm, csp_hbm, dtab_hbm, ctab_hbm, out_hbm,
             idx_v, ctab_v, sbuf, gbuf, abuf, cbuf, gsem, ssem, osem0, osem1):
    wid = lax.axis_index("s") * _NC + lax.axis_index("c")
    p0 = wid * _PW
    bsel = p0 // _S
    sbase = p0 % _S
    osem = (osem0, osem1)

    pltpu.sync_copy(idx_hbm.at[pl.ds(p0 * 8, _PW * 8)], idx_v)
    pltpu.sync_copy(ctab_hbm, ctab_v)

    def prefetch(s, nb):
        pltpu.async_copy(csp_hbm.at[pl.ds(p0 + s * _PP, _PP)],
                         sbuf.at[nb], ssem)

    def do_step(s, nb):
        pltpu.make_async_copy(csp_hbm.at[pl.ds(0, _PP)],
                              sbuf.at[nb], ssem).wait()

        ghs = []
        for pp in range(_PP):
            off = pl.multiple_of((s * _PP + pp) * 8, 8)
            ghs.append(pltpu.async_copy(
                dtab_hbm.at[idx_v.at[pl.ds(off, _ND)]],
                gbuf.at[nb, pp], gsem))

        for jj in range(_TAIL):
            j = jj + (_HEAD - _ND)
            sp = [sbuf[nb, pp, j, pl.ds(0, _L)] for pp in range(_PP)]
            rows = [ctab_v[j, pl.ds(t * _L, _L)] for t in range(_G)]
            for g in range(_NK // _G):
                base = g * _G
                prods = [[sp[pp] * rows[t] for pp in range(_PP)]
                         for t in range(_G)]
                if g + 1 < _NK // _G:
                    rows = [ctab_v[j, pl.ds((base + _G + t) * _L, _L)]
                            for t in range(_G)]
                for t in range(_G):
                    for pp in range(_PP):
                        cbuf[nb, pp, jj, pl.ds((base + t) * _L, _L)] = \
                            prods[t][pp]

        for gh in ghs:
            gh.wait()

        sph = [[sbuf[nb, pp, j, pl.ds(0, _L)] for j in range(_HEAD - _ND)]
               for pp in range(_PP)]
        for kg in range(_NK // _G):
            for t in range(_G):
                ks = pl.ds((kg * _G + t) * _L, _L)
                rows = [ctab_v[j, ks] for j in range(_HEAD - _ND)]
                gcp = [[gbuf[nb, pp, r, ks] for r in range(_ND)]
                       for pp in range(_PP)]
                for j in range(_HEAD - _ND):
                    for pp in range(_PP):
                        abuf[nb, pp, _ND + j, ks] = sph[pp][j] * rows[j]
                for pp in range(_PP):
                    for r in range(_ND):
                        abuf[nb, pp, r, ks] = gcp[pp][r]

        spos = sbase + s * _PP
        for pp in range(_PP):
            pltpu.async_copy(abuf.at[nb, pp],
                             out_hbm.at[bsel, spos + pp, pl.ds(0, _HEAD)],
                             osem[nb])
            pltpu.async_copy(cbuf.at[nb, pp],
                             out_hbm.at[bsel, spos + pp, pl.ds(_HEAD, _TAIL)],
                             osem[nb])

        prefetch(jnp.minimum(s + 2, _STEPS - 1), nb)

    def drain_out(nb):
        pltpu.make_async_copy(out_hbm.at[0, pl.ds(0, _PP), pl.ds(0, _HEAD)],
                              abuf.at[nb], osem[nb]).wait()
        pltpu.make_async_copy(out_hbm.at[0, pl.ds(0, _PP), pl.ds(_HEAD, _TAIL)],
                              cbuf.at[nb], osem[nb]).wait()

    def drain_splat(nb):
        pltpu.make_async_copy(csp_hbm.at[pl.ds(0, _PP)],
                              sbuf.at[nb], ssem).wait()

    prefetch(0, 0)
    prefetch(1, 1)
    for nb in range(2):
        pltpu.async_copy(out_hbm.at[0, pl.ds(0, _PP), pl.ds(0, _HEAD)],
                         abuf.at[nb], osem[nb])
        pltpu.async_copy(out_hbm.at[0, pl.ds(0, _PP), pl.ds(_HEAD, _TAIL)],
                         cbuf.at[nb], osem[nb])

    def outer(s2, c):
        for nb in range(2):
            drain_out(nb)
            do_step(s2 * 2 + nb, nb)
        return c
    lax.fori_loop(0, _STEPS // 2, outer, 0)
    drain_out(0)
    drain_out(1)
    drain_splat(0)
    drain_splat(1)


@jax.jit
def _sc_call(flat_idx8, cont_splat, disc_table, cont_table):
    mesh = plsc.VectorSubcoreMesh(core_axis_name="c", subcore_axis_name="s")
    f = functools.partial(
        pl.kernel, _sc_body, mesh=mesh,
        out_type=jax.ShapeDtypeStruct((_N // _S, _S, _NROW, _DIM), jnp.float32),
        scratch_types=[
            pltpu.VMEM((_PW * 8,), jnp.int32),
            pltpu.VMEM((_NCONT, _DIM), jnp.float32),
            pltpu.VMEM((2, _PP, _NCONT, _L), jnp.float32),
            pltpu.VMEM((2, _PP, _ND, _DIM), jnp.float32),
            pltpu.VMEM((2, _PP, _HEAD, _DIM), jnp.float32),
            pltpu.VMEM((2, _PP, _TAIL, _DIM), jnp.float32),
            pltpu.SemaphoreType.DMA,
            pltpu.SemaphoreType.DMA,
            pltpu.SemaphoreType.DMA,
            pltpu.SemaphoreType.DMA,
        ],
    )()
    return f(flat_idx8, cont_splat, disc_table, cont_table)


def kernel(discrete_actions, continuous_actions, disc_table, cont_table, offsets):
    b, s, n_disc = discrete_actions.shape
    n_cont = continuous_actions.shape[-1]
    dim = disc_table.shape[-1]
    n = b * s
    flat_idx = (discrete_actions + offsets[None, None, :]).reshape(n, n_disc)
    flat_idx8 = jnp.pad(flat_idx, ((0, 0), (0, 8 - n_disc))).reshape(n * 8)
    cont_splat = jnp.broadcast_to(
        continuous_actions.reshape(n, n_cont)[:, :, None], (n, n_cont, _L))
    out = _sc_call(flat_idx8, cont_splat, disc_table, cont_table)
    return out.reshape(b, s, n_disc + n_cont, dim)


# speedup vs baseline: 1.7538x; 1.7538x over previous
import jax
import jax.numpy as jnp
from jax.experimental import pallas as pl


def _body(idx_ref, cont_ref, disc_tab_ref, cont_tab_ref, out_ref):
    idx = idx_ref[...]
    r = idx.shape[0]
    vocab = disc_tab_ref.shape[0]
    iota = jax.lax.broadcasted_iota(jnp.int32, (r, 4, vocab), 2)
    one_hot = (idx[:, :, None] == iota).astype(jnp.float32)
    disc = jax.lax.dot_general(
        one_hot, disc_tab_ref[...],
        dimension_numbers=(((2,), (0,)), ((), ())),
        preferred_element_type=jnp.float32,
    )
    cont = cont_ref[...][:, :, None] * cont_tab_ref[...][None, :, :]
    out_ref[...] = jnp.concatenate([disc, cont], axis=1)


def kernel(discrete_actions, continuous_actions, disc_table, cont_table, offsets):
    b, s, n_disc = discrete_actions.shape
    n_cont = continuous_actions.shape[-1]
    dim = disc_table.shape[-1]
    n = b * s
    flat_idx = (discrete_actions + offsets[None, None, :]).reshape(n, n_disc)
    cont = continuous_actions.reshape(n, n_cont)

    R = 128
    grid = (n // R,)
    out = pl.pallas_call(
        _body,
        grid=grid,
        in_specs=[
            pl.BlockSpec((R, n_disc), lambda i: (i, 0)),
            pl.BlockSpec((R, n_cont), lambda i: (i, 0)),
            pl.BlockSpec(disc_table.shape, lambda i: (0, 0)),
            pl.BlockSpec(cont_table.shape, lambda i: (0, 0)),
        ],
        out_specs=pl.BlockSpec((R, n_disc + n_cont, dim), lambda i: (i, 0, 0)),
        out_shape=jax.ShapeDtypeStruct((n, n_disc + n_cont, dim), jnp.float32),
    )(flat_idx, cont, disc_table, cont_table)
    return out.reshape(b, s, n_disc + n_cont, dim)
